# Initial kernel scaffold; baseline (speedup 1.0000x reference)
import jax
import jax.numpy as jnp
from jax.experimental import pallas as pl


def kernel(X, params):
    raise NotImplementedError("write your pallas kernel here")


# speedup vs baseline: 109.0905x; 109.0905x over previous
import functools

import jax
import jax.numpy as jnp
from jax import lax
from jax.experimental import pallas as pl
from jax.experimental.pallas import tpu as pltpu
from jax.experimental.pallas import tpu_sc as plsc

B, D, ARITY, OUT = 1024, 512, 8, 16
NMASK = 1 << ARITY
NC, NS, L = 2, 16, 16
NW = NC * NS
D_PER_W = D // NW
GROUPS = B // L

_MASK_ALL4 = (NMASK - 1) * OUT
NEG_INF = float("-inf")


def _body(xt_hbm, pr_hbm, out_hbm, xv, pv, ov):
    wid = lax.axis_index("s") * NC + lax.axis_index("c")
    iota = lax.iota(jnp.int32, L)
    o_iota = iota * OUT

    def per_d(j, _):
        d = wid * D_PER_W + j
        pltpu.sync_copy(xt_hbm.at[d], xv)
        pltpu.sync_copy(pr_hbm.at[d], pv)

        def per_group(g, _):
            b0 = g * L
            x = [xv[pl.ds(a * B + b0, L)] for a in range(ARITY)]
            zero = jnp.zeros((L,), jnp.int32)
            macc4 = [zero for _ in range(ARITY)]
            prev = [jnp.full((L,), NEG_INF) for _ in range(ARITY)]
            for a2 in range(ARITY):
                for a in range(a2 + 1, ARITY):
                    le = x[a2] <= x[a]
                    macc4[a] = macc4[a] + jnp.where(
                        le, jnp.int32((1 << a2) * OUT), 0)
                    macc4[a2] = macc4[a2] + jnp.where(
                        le, 0, jnp.int32((1 << a) * OUT))
                    prev[a] = jnp.maximum(prev[a], jnp.where(le, x[a2], NEG_INF))
                    prev[a2] = jnp.maximum(prev[a2], jnp.where(le, NEG_INF, x[a]))
            midx = [_MASK_ALL4 - m for m in macc4]
            coef = [x[a] - jnp.where(macc4[a] == 0, jnp.float32(0), prev[a])
                    for a in range(ARITY)]
            acc = [jnp.zeros((L,), jnp.float32) for _ in range(OUT)]
            for k in range(ARITY):
                for l in range(OUT):
                    row = plsc.load_gather(pv, [midx[k] + l])
                    acc[l] = acc[l] + coef[k] * row
            obase = o_iota + b0 * OUT
            for l in range(OUT):
                plsc.store_scatter(ov, [obase + l], acc[l])
            return 0

        lax.fori_loop(0, GROUPS, per_group, 0)
        pltpu.sync_copy(ov, out_hbm.at[d])
        return 0

    lax.fori_loop(0, D_PER_W, per_d, 0)


@jax.jit
def kernel(X, params):
    xt = jnp.transpose(X, (1, 2, 0)).reshape(D, ARITY * B)
    pr = params.reshape(D, NMASK * OUT)
    run = pl.kernel(
        _body,
        out_type=jax.ShapeDtypeStruct((D, B * OUT), jnp.float32),
        mesh=plsc.VectorSubcoreMesh(core_axis_name="c", subcore_axis_name="s"),
        compiler_params=pltpu.CompilerParams(needs_layout_passes=False),
        scratch_types=[
            pltpu.VMEM((ARITY * B,), jnp.float32),
            pltpu.VMEM((NMASK * OUT,), jnp.float32),
            pltpu.VMEM((B * OUT,), jnp.float32),
        ],
    )
    out_t = run(xt, pr)
    return jnp.transpose(out_t.reshape(D, B, OUT), (1, 0, 2))
